# Initial kernel scaffold; baseline (speedup 1.0000x reference)
import jax
import jax.numpy as jnp
from jax.experimental import pallas as pl


def kernel(pos, atom_type, batch, params):
    raise NotImplementedError("write your pallas kernel here")


# speedup vs baseline: 13.2105x; 13.2105x over previous
import jax
import jax.numpy as jnp
from jax.experimental import pallas as pl
from jax.experimental.pallas import tpu as pltpu

_B, _NPG, _K, _N, _D, _CUTOFF = 20, 500, 32, 10000, 128, 10.0
_NEXP = 4
_NL = 2
_P = 512
_ATP = 256


def _graph_body(posp_ref, post_ref, oha_ref, emb_ref, ew_ref, eb_ref,
                sp_ref, out_ref, d2_ref, nbr_ref, ewm_ref):
    f32 = jnp.float32
    lane_p = jax.lax.broadcasted_iota(jnp.int32, (_P, _P), 1).astype(f32)
    sub_p = jax.lax.broadcasted_iota(jnp.int32, (_P, _P), 0).astype(f32)

    bad = (lane_p == sub_p) | (lane_p >= _NPG)
    d2 = jnp.where(bad, f32(1e9), f32(0.0))
    for c in range(3):
        diff = posp_ref[:, c:c + 1] - post_ref[c:c + 1, :]
        d2 = d2 + diff * diff
    d2_ref[...] = d2

    ewm_ref[...] = jnp.zeros((_P, 128), f32)
    for k in range(_K):
        dd = d2_ref[...]
        m = jnp.min(dd, axis=1, keepdims=True)
        idxf = jnp.min(jnp.where(dd == m, lane_p, f32(1e9)), axis=1,
                       keepdims=True)
        d2_ref[...] = jnp.where(lane_p == idxf, f32(3e9), dd)
        nbr_ref[:, k:k + 1] = idxf
        ewm_ref[:, k:k + 1] = jnp.sqrt(jnp.maximum(m, 0.0) + 1e-12)

    row_i = jax.lax.broadcasted_iota(jnp.int32, (_P, 128), 0).astype(f32)
    lane_i = jax.lax.broadcasted_iota(jnp.int32, (_P, 128), 1).astype(f32)
    rowmask = (row_i < _NPG).astype(f32)
    emask = rowmask * (lane_i < _K).astype(f32)
    ecnt = f32(_NPG * _K)

    ewm = ewm_ref[...]
    mean = jnp.sum(ewm * emask) / ecnt
    cen = ewm - sp_ref[2, 2] * mean
    var = jnp.sum(cen * cen * emask) / ecnt
    ea = sp_ref[2, 0] * cen * jax.lax.rsqrt(var + 1e-5) + sp_ref[2, 1]
    ewt = (_CUTOFF - ewm) / _CUTOFF

    h = jnp.dot(oha_ref[...], emb_ref[...], preferred_element_type=f32)

    attsums = []
    for l in range(_NL):
        logits = [sp_ref[l, 96 + e] + jnp.zeros((_P, 128), f32)
                  for e in range(_NEXP)]
        for j in range(16):
            h1 = jnp.maximum(ea * sp_ref[l, j] + sp_ref[l, 16 + j], 0.0)
            for e in range(_NEXP):
                logits[e] = logits[e] + h1 * sp_ref[l, 32 + j * 4 + e]
        mx = jnp.maximum(jnp.maximum(logits[0], logits[1]),
                         jnp.maximum(logits[2], logits[3]))
        ex = [jnp.exp(lg - mx) for lg in logits]
        den = ex[0] + ex[1] + ex[2] + ex[3]
        att = [v / den for v in ex]
        attsums.append([jnp.sum(a * emask) for a in att])
        att = [a * ewt for a in att]

        agg = [jnp.zeros((_P, _D), f32) for _ in range(_NEXP)]
        for k in range(_K):
            idxc = nbr_ref[:, k:k + 1]
            oh = (lane_p == idxc).astype(f32)
            g = jnp.dot(oh, h, preferred_element_type=f32)
            for e in range(_NEXP):
                agg[e] = agg[e] + att[e][:, k:k + 1] * g

        hn = jnp.zeros((_P, _D), f32)
        for e in range(_NEXP):
            r = l * 8 + e * 2
            t = jnp.maximum(
                jnp.dot(agg[e], ew_ref[r], preferred_element_type=f32)
                + eb_ref[r:r + 1, :], 0.0)
            hn = hn + jnp.dot(t, ew_ref[r + 1], preferred_element_type=f32) \
                + eb_ref[r + 1:r + 2, :]
        h = hn

        if l + 1 < _NL:
            mean_r = jnp.sum(h * rowmask, axis=0, keepdims=True) / f32(_NPG)
            cen_n = h - eb_ref[18:19, :] * mean_r
            var_r = jnp.sum(cen_n * cen_n * rowmask, axis=0,
                            keepdims=True) / f32(_NPG)
            h = eb_ref[16:17, :] * cen_n * jax.lax.rsqrt(var_r + 1e-5) \
                + eb_ref[17:18, :]
            h = jnp.tanh(h)

    hg = jnp.sum(h * rowmask) / f32(_NPG * _D)

    sub8 = jax.lax.broadcasted_iota(jnp.int32, (8, 128), 0)
    lane8 = jax.lax.broadcasted_iota(jnp.int32, (8, 128), 1)
    o = jnp.where((sub8 == 0) & (lane8 == 0), hg, f32(0.0))
    for l in range(_NL):
        for e in range(_NEXP):
            o = jnp.where((sub8 == l + 1) & (lane8 == e), attsums[l][e], o)
    out_ref[...] = o


def kernel(pos, atom_type, batch, params):
    f32 = jnp.float32
    pos_r = pos.reshape(_B, _NPG, 3).astype(f32)
    posp = jnp.zeros((_B, _P, 128), f32).at[:, :_NPG, :3].set(pos_r)
    posp = posp.reshape(_B * _P, 128)
    post = jnp.zeros((_B, 128, _P), f32).at[:, :3, :_NPG].set(
        pos_r.transpose(0, 2, 1)).reshape(_B * 128, _P)
    oha = jax.nn.one_hot(atom_type, _ATP, dtype=f32).reshape(_B, _NPG, _ATP)
    oha = jnp.pad(oha, ((0, 0), (0, _P - _NPG), (0, 0))).reshape(_B * _P, _ATP)
    emb = jnp.pad(params["atom_emb"].astype(f32), ((0, _ATP - 200), (0, 0)))

    layers = params["layers"]
    ew_rows, eb_rows = [], []
    for l in range(_NL):
        for e in range(_NEXP):
            exp = layers[l]["experts"][e]
            ew_rows += [exp["W1"].astype(f32), exp["W2"].astype(f32)]
            eb_rows += [exp["b1"].astype(f32), exp["b2"].astype(f32)]
    EW = jnp.stack(ew_rows)
    EB = jnp.stack(eb_rows
                   + [layers[0]["gn_g"].astype(f32),
                      layers[0]["gn_b"].astype(f32),
                      layers[0]["gn_ms"].astype(f32)]
                   + [jnp.zeros((_D,), f32)] * 5)

    sp = jnp.zeros((8, 128), f32)
    for l in range(_NL):
        L = layers[l]
        sp = sp.at[l, 0:16].set(L["eW1"].reshape(16).astype(f32))
        sp = sp.at[l, 16:32].set(L["eb1"].astype(f32))
        sp = sp.at[l, 32:96].set(L["eW2"].reshape(64).astype(f32))
        sp = sp.at[l, 96:100].set(L["eb2"].astype(f32))
    sp = sp.at[2, 0].set(params["dn_g"][0]) \
           .at[2, 1].set(params["dn_b"][0]) \
           .at[2, 2].set(params["dn_ms"][0])

    out = pl.pallas_call(
        _graph_body,
        grid=(_B,),
        in_specs=[
            pl.BlockSpec((_P, 128), lambda b: (b, 0)),
            pl.BlockSpec((128, _P), lambda b: (b, 0)),
            pl.BlockSpec((_P, _ATP), lambda b: (b, 0)),
            pl.BlockSpec((_ATP, 128), lambda b: (0, 0)),
            pl.BlockSpec((16, 128, 128), lambda b: (0, 0, 0)),
            pl.BlockSpec((24, 128), lambda b: (0, 0)),
            pl.BlockSpec(memory_space=pltpu.SMEM),
        ],
        out_specs=pl.BlockSpec((8, 128), lambda b: (b, 0)),
        out_shape=jax.ShapeDtypeStruct((_B * 8, 128), f32),
        scratch_shapes=[
            pltpu.VMEM((_P, _P), f32),
            pltpu.VMEM((_P, 128), f32),
            pltpu.VMEM((_P, 128), f32),
        ],
    )(posp, post, oha, emb, EW, EB, sp)

    outr = out.reshape(_B, 8, 128)
    hg = outr[:, 0, 0]
    means = outr[:, 1:1 + _NL, :_NEXP].sum(axis=0) / f32(_B * _NPG * _K)
    lb_layers = jnp.sum(means * means, axis=1) * _NEXP
    total_lb = jnp.sum(lb_layers) / _NL * jnp.float32(0.1)
    return hg, total_lb


# speedup vs baseline: 13.4316x; 1.0167x over previous
import jax
import jax.numpy as jnp
from jax.experimental import pallas as pl
from jax.experimental.pallas import tpu as pltpu

_B, _NPG, _K, _N, _D, _CUTOFF = 20, 500, 32, 10000, 128, 10.0
_NEXP = 4
_NL = 2
_P = 512
_ATP = 256


def _graph_body(posp_ref, post_ref, oha_ref, emb_ref, ew_ref, eb_ref,
                sp_ref, out_ref, d2_ref, oh_ref, ewm_ref):
    f32 = jnp.float32
    i32 = jnp.int32
    lane_pi = jax.lax.broadcasted_iota(i32, (_P, _P), 1)
    sub_pi = jax.lax.broadcasted_iota(i32, (_P, _P), 0)

    bad = (lane_pi == sub_pi) | (lane_pi >= _NPG)
    d2 = jnp.where(bad, f32(1e9), f32(0.0))
    for c in range(3):
        diff = posp_ref[:, c:c + 1] - post_ref[c:c + 1, :]
        d2 = d2 + diff * diff
    d2_ref[...] = d2
    lane_p = lane_pi.astype(f32)

    ewm_ref[...] = jnp.zeros((_P, 128), f32)
    for k in range(_K):
        dd = d2_ref[...]
        m = jnp.min(dd, axis=1, keepdims=True)
        idxf = jnp.min(jnp.where(dd == m, lane_p, f32(1e9)), axis=1,
                       keepdims=True)
        hit = lane_p == idxf
        oh_ref[k] = hit.astype(f32)
        d2_ref[...] = jnp.where(hit, f32(3e9), dd)
        ewm_ref[:, k:k + 1] = jnp.sqrt(jnp.maximum(m, 0.0) + 1e-12)

    row_i = jax.lax.broadcasted_iota(jnp.int32, (_P, 128), 0).astype(f32)
    lane_i = jax.lax.broadcasted_iota(jnp.int32, (_P, 128), 1).astype(f32)
    rowmask = (row_i < _NPG).astype(f32)
    emask = rowmask * (lane_i < _K).astype(f32)
    ecnt = f32(_NPG * _K)

    ewm = ewm_ref[...]
    mean = jnp.sum(ewm * emask) / ecnt
    cen = ewm - sp_ref[2, 2] * mean
    var = jnp.sum(cen * cen * emask) / ecnt
    ea = sp_ref[2, 0] * cen * jax.lax.rsqrt(var + 1e-5) + sp_ref[2, 1]
    ewt = (_CUTOFF - ewm) / _CUTOFF

    h = jnp.dot(oha_ref[...], emb_ref[...], preferred_element_type=f32)

    attsums = []
    for l in range(_NL):
        logits = [sp_ref[l, 96 + e] + jnp.zeros((_P, 128), f32)
                  for e in range(_NEXP)]
        for j in range(16):
            h1 = jnp.maximum(ea * sp_ref[l, j] + sp_ref[l, 16 + j], 0.0)
            for e in range(_NEXP):
                logits[e] = logits[e] + h1 * sp_ref[l, 32 + j * 4 + e]
        mx = jnp.maximum(jnp.maximum(logits[0], logits[1]),
                         jnp.maximum(logits[2], logits[3]))
        ex = [jnp.exp(lg - mx) for lg in logits]
        den = ex[0] + ex[1] + ex[2] + ex[3]
        att = [v / den for v in ex]
        attsums.append([jnp.sum(a * emask) for a in att])
        att = [a * ewt for a in att]

        agg = [jnp.zeros((_P, _D), f32) for _ in range(_NEXP)]
        for k in range(_K):
            g = jnp.dot(oh_ref[k], h, preferred_element_type=f32)
            for e in range(_NEXP):
                agg[e] = agg[e] + att[e][:, k:k + 1] * g

        hn = jnp.zeros((_P, _D), f32)
        for e in range(_NEXP):
            r = l * 8 + e * 2
            t = jnp.maximum(
                jnp.dot(agg[e], ew_ref[r], preferred_element_type=f32)
                + eb_ref[r:r + 1, :], 0.0)
            hn = hn + jnp.dot(t, ew_ref[r + 1], preferred_element_type=f32) \
                + eb_ref[r + 1:r + 2, :]
        h = hn

        if l + 1 < _NL:
            mean_r = jnp.sum(h * rowmask, axis=0, keepdims=True) / f32(_NPG)
            cen_n = h - eb_ref[18:19, :] * mean_r
            var_r = jnp.sum(cen_n * cen_n * rowmask, axis=0,
                            keepdims=True) / f32(_NPG)
            h = eb_ref[16:17, :] * cen_n * jax.lax.rsqrt(var_r + 1e-5) \
                + eb_ref[17:18, :]
            h = jnp.tanh(h)

    hg = jnp.sum(h * rowmask) / f32(_NPG * _D)

    sub8 = jax.lax.broadcasted_iota(jnp.int32, (8, 128), 0)
    lane8 = jax.lax.broadcasted_iota(jnp.int32, (8, 128), 1)
    o = jnp.where((sub8 == 0) & (lane8 == 0), hg, f32(0.0))
    for l in range(_NL):
        for e in range(_NEXP):
            o = jnp.where((sub8 == l + 1) & (lane8 == e), attsums[l][e], o)
    out_ref[...] = o


def kernel(pos, atom_type, batch, params):
    f32 = jnp.float32
    pos_r = pos.reshape(_B, _NPG, 3).astype(f32)
    posp = jnp.zeros((_B, _P, 128), f32).at[:, :_NPG, :3].set(pos_r)
    posp = posp.reshape(_B * _P, 128)
    post = jnp.zeros((_B, 128, _P), f32).at[:, :3, :_NPG].set(
        pos_r.transpose(0, 2, 1)).reshape(_B * 128, _P)
    oha = jax.nn.one_hot(atom_type, _ATP, dtype=f32).reshape(_B, _NPG, _ATP)
    oha = jnp.pad(oha, ((0, 0), (0, _P - _NPG), (0, 0))).reshape(_B * _P, _ATP)
    emb = jnp.pad(params["atom_emb"].astype(f32), ((0, _ATP - 200), (0, 0)))

    layers = params["layers"]
    ew_rows, eb_rows = [], []
    for l in range(_NL):
        for e in range(_NEXP):
            exp = layers[l]["experts"][e]
            ew_rows += [exp["W1"].astype(f32), exp["W2"].astype(f32)]
            eb_rows += [exp["b1"].astype(f32), exp["b2"].astype(f32)]
    EW = jnp.stack(ew_rows)
    EB = jnp.stack(eb_rows
                   + [layers[0]["gn_g"].astype(f32),
                      layers[0]["gn_b"].astype(f32),
                      layers[0]["gn_ms"].astype(f32)]
                   + [jnp.zeros((_D,), f32)] * 5)

    sp = jnp.zeros((8, 128), f32)
    for l in range(_NL):
        L = layers[l]
        sp = sp.at[l, 0:16].set(L["eW1"].reshape(16).astype(f32))
        sp = sp.at[l, 16:32].set(L["eb1"].astype(f32))
        sp = sp.at[l, 32:96].set(L["eW2"].reshape(64).astype(f32))
        sp = sp.at[l, 96:100].set(L["eb2"].astype(f32))
    sp = sp.at[2, 0].set(params["dn_g"][0]) \
           .at[2, 1].set(params["dn_b"][0]) \
           .at[2, 2].set(params["dn_ms"][0])

    out = pl.pallas_call(
        _graph_body,
        grid=(_B,),
        in_specs=[
            pl.BlockSpec((_P, 128), lambda b: (b, 0)),
            pl.BlockSpec((128, _P), lambda b: (b, 0)),
            pl.BlockSpec((_P, _ATP), lambda b: (b, 0)),
            pl.BlockSpec((_ATP, 128), lambda b: (0, 0)),
            pl.BlockSpec((16, 128, 128), lambda b: (0, 0, 0)),
            pl.BlockSpec((24, 128), lambda b: (0, 0)),
            pl.BlockSpec(memory_space=pltpu.SMEM),
        ],
        out_specs=pl.BlockSpec((8, 128), lambda b: (b, 0)),
        out_shape=jax.ShapeDtypeStruct((_B * 8, 128), f32),
        scratch_shapes=[
            pltpu.VMEM((_P, _P), f32),
            pltpu.VMEM((_K, _P, _P), f32),
            pltpu.VMEM((_P, 128), f32),
        ],
    )(posp, post, oha, emb, EW, EB, sp)

    outr = out.reshape(_B, 8, 128)
    hg = outr[:, 0, 0]
    means = outr[:, 1:1 + _NL, :_NEXP].sum(axis=0) / f32(_B * _NPG * _K)
    lb_layers = jnp.sum(means * means, axis=1) * _NEXP
    total_lb = jnp.sum(lb_layers) / _NL * jnp.float32(0.1)
    return hg, total_lb


# speedup vs baseline: 16.8122x; 1.2517x over previous
import jax
import jax.numpy as jnp
from jax.experimental import pallas as pl
from jax.experimental.pallas import tpu as pltpu

_B, _NPG, _K, _N, _D, _CUTOFF = 20, 500, 32, 10000, 128, 10.0
_NEXP = 4
_NL = 2
_P = 512
_ATP = 256


def _graph_body(posp_ref, post_ref, oha_ref, emb_ref, ew_ref, eb_ref,
                sp_ref, bx_ref, out_ref, d2_ref, oh_ref, ewm_ref):
    f32 = jnp.float32
    i32 = jnp.int32
    lane_pi = jax.lax.broadcasted_iota(i32, (_P, _P), 1)
    sub_pi = jax.lax.broadcasted_iota(i32, (_P, _P), 0)

    bad = (lane_pi == sub_pi) | (lane_pi >= _NPG)
    d2 = jnp.where(bad, f32(1e9), f32(0.0))
    for c in range(3):
        diff = posp_ref[:, c:c + 1] - post_ref[c:c + 1, :]
        d2 = d2 + diff * diff
    d2_ref[...] = d2
    lane_p = lane_pi.astype(f32)

    ewm_ref[...] = jnp.zeros((_P, 128), f32)
    for k in range(_K):
        dd = d2_ref[...]
        m = jnp.min(dd, axis=1, keepdims=True)
        idxf = jnp.min(jnp.where(dd == m, lane_p, f32(1e9)), axis=1,
                       keepdims=True)
        hit = lane_p == idxf
        oh_ref[k] = hit.astype(f32)
        d2_ref[...] = jnp.where(hit, f32(3e9), dd)
        ewm_ref[:, k:k + 1] = jnp.sqrt(jnp.maximum(m, 0.0) + 1e-12)

    row_i = jax.lax.broadcasted_iota(jnp.int32, (_P, 128), 0).astype(f32)
    lane_i = jax.lax.broadcasted_iota(jnp.int32, (_P, 128), 1).astype(f32)
    rowmask = (row_i < _NPG).astype(f32)
    emask = rowmask * (lane_i < _K).astype(f32)
    ecnt = f32(_NPG * _K)

    ewm = ewm_ref[...]
    mean = jnp.sum(ewm * emask) / ecnt
    cen = ewm - sp_ref[2, 2] * mean
    var = jnp.sum(cen * cen * emask) / ecnt
    ea = sp_ref[2, 0] * cen * jax.lax.rsqrt(var + 1e-5) + sp_ref[2, 1]
    ewt = (_CUTOFF - ewm) / _CUTOFF

    h = jnp.dot(oha_ref[...], emb_ref[...], preferred_element_type=f32)

    attsums = []
    for l in range(_NL):
        logits = [sp_ref[l, 96 + e] + jnp.zeros((_P, 128), f32)
                  for e in range(_NEXP)]
        for j in range(16):
            h1 = jnp.maximum(ea * sp_ref[l, j] + sp_ref[l, 16 + j], 0.0)
            for e in range(_NEXP):
                logits[e] = logits[e] + h1 * sp_ref[l, 32 + j * 4 + e]
        mx = jnp.maximum(jnp.maximum(logits[0], logits[1]),
                         jnp.maximum(logits[2], logits[3]))
        ex = [jnp.exp(lg - mx) for lg in logits]
        den = ex[0] + ex[1] + ex[2] + ex[3]
        att = [v / den for v in ex]
        attsums.append([jnp.sum(a * emask) for a in att])
        att = [a * ewt for a in att]

        attw = [jnp.dot(att[e], bx_ref[...], preferred_element_type=f32)
                for e in range(_NEXP)]

        agg = [jnp.zeros((_P, _D), f32) for _ in range(_NEXP)]
        for k in range(_K):
            g = jnp.dot(oh_ref[k], h, preferred_element_type=f32)
            for e in range(_NEXP):
                agg[e] = agg[e] + attw[e][:, _D * k:_D * (k + 1)] * g

        hn = jnp.zeros((_P, _D), f32)
        for e in range(_NEXP):
            r = l * 8 + e * 2
            t = jnp.maximum(
                jnp.dot(agg[e], ew_ref[r], preferred_element_type=f32)
                + eb_ref[r:r + 1, :], 0.0)
            hn = hn + jnp.dot(t, ew_ref[r + 1], preferred_element_type=f32) \
                + eb_ref[r + 1:r + 2, :]
        h = hn

        if l + 1 < _NL:
            mean_r = jnp.sum(h * rowmask, axis=0, keepdims=True) / f32(_NPG)
            cen_n = h - eb_ref[18:19, :] * mean_r
            var_r = jnp.sum(cen_n * cen_n * rowmask, axis=0,
                            keepdims=True) / f32(_NPG)
            h = eb_ref[16:17, :] * cen_n * jax.lax.rsqrt(var_r + 1e-5) \
                + eb_ref[17:18, :]
            h = jnp.tanh(h)

    hg = jnp.sum(h * rowmask) / f32(_NPG * _D)

    sub8 = jax.lax.broadcasted_iota(jnp.int32, (8, 128), 0)
    lane8 = jax.lax.broadcasted_iota(jnp.int32, (8, 128), 1)
    o = jnp.where((sub8 == 0) & (lane8 == 0), hg, f32(0.0))
    for l in range(_NL):
        for e in range(_NEXP):
            o = jnp.where((sub8 == l + 1) & (lane8 == e), attsums[l][e], o)
    out_ref[...] = o


def kernel(pos, atom_type, batch, params):
    f32 = jnp.float32
    pos_r = pos.reshape(_B, _NPG, 3).astype(f32)
    posp = jnp.zeros((_B, _P, 128), f32).at[:, :_NPG, :3].set(pos_r)
    posp = posp.reshape(_B * _P, 128)
    post = jnp.zeros((_B, 128, _P), f32).at[:, :3, :_NPG].set(
        pos_r.transpose(0, 2, 1)).reshape(_B * 128, _P)
    oha = jax.nn.one_hot(atom_type, _ATP, dtype=f32).reshape(_B, _NPG, _ATP)
    oha = jnp.pad(oha, ((0, 0), (0, _P - _NPG), (0, 0))).reshape(_B * _P, _ATP)
    emb = jnp.pad(params["atom_emb"].astype(f32), ((0, _ATP - 200), (0, 0)))

    layers = params["layers"]
    ew_rows, eb_rows = [], []
    for l in range(_NL):
        for e in range(_NEXP):
            exp = layers[l]["experts"][e]
            ew_rows += [exp["W1"].astype(f32), exp["W2"].astype(f32)]
            eb_rows += [exp["b1"].astype(f32), exp["b2"].astype(f32)]
    EW = jnp.stack(ew_rows)
    EB = jnp.stack(eb_rows
                   + [layers[0]["gn_g"].astype(f32),
                      layers[0]["gn_b"].astype(f32),
                      layers[0]["gn_ms"].astype(f32)]
                   + [jnp.zeros((_D,), f32)] * 5)

    sp = jnp.zeros((8, 128), f32)
    for l in range(_NL):
        L = layers[l]
        sp = sp.at[l, 0:16].set(L["eW1"].reshape(16).astype(f32))
        sp = sp.at[l, 16:32].set(L["eb1"].astype(f32))
        sp = sp.at[l, 32:96].set(L["eW2"].reshape(64).astype(f32))
        sp = sp.at[l, 96:100].set(L["eb2"].astype(f32))
    sp = sp.at[2, 0].set(params["dn_g"][0]) \
           .at[2, 1].set(params["dn_b"][0]) \
           .at[2, 2].set(params["dn_ms"][0])

    bx = jnp.zeros((128, _K * _D), f32).at[:_K, :].set(
        jnp.kron(jnp.eye(_K, dtype=f32), jnp.ones((1, _D), f32)))

    out = pl.pallas_call(
        _graph_body,
        grid=(_B,),
        in_specs=[
            pl.BlockSpec((_P, 128), lambda b: (b, 0)),
            pl.BlockSpec((128, _P), lambda b: (b, 0)),
            pl.BlockSpec((_P, _ATP), lambda b: (b, 0)),
            pl.BlockSpec((_ATP, 128), lambda b: (0, 0)),
            pl.BlockSpec((16, 128, 128), lambda b: (0, 0, 0)),
            pl.BlockSpec((24, 128), lambda b: (0, 0)),
            pl.BlockSpec(memory_space=pltpu.SMEM),
            pl.BlockSpec((128, _K * _D), lambda b: (0, 0)),
        ],
        out_specs=pl.BlockSpec((8, 128), lambda b: (b, 0)),
        out_shape=jax.ShapeDtypeStruct((_B * 8, 128), f32),
        scratch_shapes=[
            pltpu.VMEM((_P, _P), f32),
            pltpu.VMEM((_K, _P, _P), f32),
            pltpu.VMEM((_P, 128), f32),
        ],
    )(posp, post, oha, emb, EW, EB, sp, bx)

    outr = out.reshape(_B, 8, 128)
    hg = outr[:, 0, 0]
    means = outr[:, 1:1 + _NL, :_NEXP].sum(axis=0) / f32(_B * _NPG * _K)
    lb_layers = jnp.sum(means * means, axis=1) * _NEXP
    total_lb = jnp.sum(lb_layers) / _NL * jnp.float32(0.1)
    return hg, total_lb


# speedup vs baseline: 16.8202x; 1.0005x over previous
import jax
import jax.numpy as jnp
from jax.experimental import pallas as pl
from jax.experimental.pallas import tpu as pltpu

_B, _NPG, _K, _N, _D, _CUTOFF = 20, 500, 32, 10000, 128, 10.0
_NEXP = 4
_NL = 2
_P = 512
_ATP = 256


def _graph_body(posp_ref, post_ref, oha_ref, emb_ref, ew_ref, eb_ref,
                sp_ref, bx_ref, out_ref, d2_ref, oh_ref, ewm_ref):
    f32 = jnp.float32
    i32 = jnp.int32
    lane_pi = jax.lax.broadcasted_iota(i32, (_P, _P), 1)
    sub_pi = jax.lax.broadcasted_iota(i32, (_P, _P), 0)

    bad = (lane_pi == sub_pi) | (lane_pi >= _NPG)
    d2 = jnp.where(bad, f32(1e9), f32(0.0))
    for c in range(3):
        diff = posp_ref[:, c:c + 1] - post_ref[c:c + 1, :]
        d2 = d2 + diff * diff
    d2_ref[...] = d2
    lane_p = lane_pi.astype(f32)

    ewm_ref[...] = jnp.zeros((_P, 128), f32)
    for k in range(_K):
        dd = d2_ref[...]
        m = jnp.min(dd, axis=1, keepdims=True)
        idxf = jnp.min(jnp.where(dd == m, lane_p, f32(1e9)), axis=1,
                       keepdims=True)
        hit = lane_p == idxf
        oh_ref[k] = hit.astype(f32)
        d2_ref[...] = jnp.where(hit, f32(3e9), dd)
        ewm_ref[:, k:k + 1] = jnp.sqrt(jnp.maximum(m, 0.0) + 1e-12)

    row_i = jax.lax.broadcasted_iota(jnp.int32, (_P, 128), 0).astype(f32)
    lane_i = jax.lax.broadcasted_iota(jnp.int32, (_P, 128), 1).astype(f32)
    rowmask = (row_i < _NPG).astype(f32)
    emask = rowmask * (lane_i < _K).astype(f32)
    ecnt = f32(_NPG * _K)

    ewm = ewm_ref[...]
    mean = jnp.sum(ewm * emask) / ecnt
    cen = ewm - sp_ref[2, 2] * mean
    var = jnp.sum(cen * cen * emask) / ecnt
    ea = sp_ref[2, 0] * cen * jax.lax.rsqrt(var + 1e-5) + sp_ref[2, 1]
    ewt = (_CUTOFF - ewm) / _CUTOFF

    h = jnp.dot(oha_ref[...], emb_ref[...], preferred_element_type=f32)

    attsums = []
    for l in range(_NL):
        logits = [sp_ref[l, 96 + e] + jnp.zeros((_P, 128), f32)
                  for e in range(_NEXP)]
        for j in range(16):
            h1 = jnp.maximum(ea * sp_ref[l, j] + sp_ref[l, 16 + j], 0.0)
            for e in range(_NEXP):
                logits[e] = logits[e] + h1 * sp_ref[l, 32 + j * 4 + e]
        mx = jnp.maximum(jnp.maximum(logits[0], logits[1]),
                         jnp.maximum(logits[2], logits[3]))
        ex = [jnp.exp(lg - mx) for lg in logits]
        den = ex[0] + ex[1] + ex[2] + ex[3]
        att = [v / den for v in ex]
        attsums.append([jnp.sum(a * emask) for a in att])
        att = [a * ewt for a in att]

        attw = [jnp.dot(att[e], bx_ref[...], preferred_element_type=f32)
                for e in range(_NEXP)]

        agg = [jnp.zeros((_P, _D), f32) for _ in range(_NEXP)]
        for k in range(_K):
            g = jnp.dot(oh_ref[k], h, preferred_element_type=f32)
            for e in range(_NEXP):
                agg[e] = agg[e] + attw[e][:, _D * k:_D * (k + 1)] * g

        hn = jnp.zeros((_P, _D), f32)
        for e in range(_NEXP):
            r = l * 8 + e * 2
            t = jnp.maximum(
                jnp.dot(agg[e], ew_ref[r], preferred_element_type=f32)
                + eb_ref[r:r + 1, :], 0.0)
            hn = hn + jnp.dot(t, ew_ref[r + 1], preferred_element_type=f32) \
                + eb_ref[r + 1:r + 2, :]
        h = hn

        if l + 1 < _NL:
            mean_r = jnp.sum(h * rowmask, axis=0, keepdims=True) / f32(_NPG)
            cen_n = h - eb_ref[18:19, :] * mean_r
            var_r = jnp.sum(cen_n * cen_n * rowmask, axis=0,
                            keepdims=True) / f32(_NPG)
            h = eb_ref[16:17, :] * cen_n * jax.lax.rsqrt(var_r + 1e-5) \
                + eb_ref[17:18, :]
            h = jnp.tanh(h)

    hg = jnp.sum(h * rowmask) / f32(_NPG * _D)

    sub8 = jax.lax.broadcasted_iota(jnp.int32, (8, 128), 0)
    lane8 = jax.lax.broadcasted_iota(jnp.int32, (8, 128), 1)
    o = jnp.where((sub8 == 0) & (lane8 == 0), hg, f32(0.0))
    for l in range(_NL):
        for e in range(_NEXP):
            o = jnp.where((sub8 == l + 1) & (lane8 == e), attsums[l][e], o)
    out_ref[...] = o


def kernel(pos, atom_type, batch, params):
    f32 = jnp.float32
    pos_r = pos.reshape(_B, _NPG, 3).astype(f32)
    posp = jnp.zeros((_B, _P, 128), f32).at[:, :_NPG, :3].set(pos_r)
    posp = posp.reshape(_B * _P, 128)
    post = jnp.zeros((_B, 128, _P), f32).at[:, :3, :_NPG].set(
        pos_r.transpose(0, 2, 1)).reshape(_B * 128, _P)
    oha = jax.nn.one_hot(atom_type, _ATP, dtype=f32).reshape(_B, _NPG, _ATP)
    oha = jnp.pad(oha, ((0, 0), (0, _P - _NPG), (0, 0))).reshape(_B * _P, _ATP)
    emb = jnp.pad(params["atom_emb"].astype(f32), ((0, _ATP - 200), (0, 0)))

    layers = params["layers"]
    ew_rows, eb_rows = [], []
    for l in range(_NL):
        for e in range(_NEXP):
            exp = layers[l]["experts"][e]
            ew_rows += [exp["W1"].astype(f32), exp["W2"].astype(f32)]
            eb_rows += [exp["b1"].astype(f32), exp["b2"].astype(f32)]
    EW = jnp.stack(ew_rows)
    EB = jnp.stack(eb_rows
                   + [layers[0]["gn_g"].astype(f32),
                      layers[0]["gn_b"].astype(f32),
                      layers[0]["gn_ms"].astype(f32)]
                   + [jnp.zeros((_D,), f32)] * 5)

    sp = jnp.zeros((8, 128), f32)
    for l in range(_NL):
        L = layers[l]
        sp = sp.at[l, 0:16].set(L["eW1"].reshape(16).astype(f32))
        sp = sp.at[l, 16:32].set(L["eb1"].astype(f32))
        sp = sp.at[l, 32:96].set(L["eW2"].reshape(64).astype(f32))
        sp = sp.at[l, 96:100].set(L["eb2"].astype(f32))
    sp = sp.at[2, 0].set(params["dn_g"][0]) \
           .at[2, 1].set(params["dn_b"][0]) \
           .at[2, 2].set(params["dn_ms"][0])

    bx = jnp.zeros((128, _K * _D), f32).at[:_K, :].set(
        jnp.kron(jnp.eye(_K, dtype=f32), jnp.ones((1, _D), f32)))

    out = pl.pallas_call(
        _graph_body,
        grid=(_B,),
        in_specs=[
            pl.BlockSpec((_P, 128), lambda b: (b, 0)),
            pl.BlockSpec((128, _P), lambda b: (b, 0)),
            pl.BlockSpec((_P, _ATP), lambda b: (b, 0)),
            pl.BlockSpec((_ATP, 128), lambda b: (0, 0)),
            pl.BlockSpec((16, 128, 128), lambda b: (0, 0, 0)),
            pl.BlockSpec((24, 128), lambda b: (0, 0)),
            pl.BlockSpec(memory_space=pltpu.SMEM),
            pl.BlockSpec((128, _K * _D), lambda b: (0, 0)),
        ],
        out_specs=pl.BlockSpec((8, 128), lambda b: (b, 0)),
        out_shape=jax.ShapeDtypeStruct((_B * 8, 128), f32),
        scratch_shapes=[
            pltpu.VMEM((_P, _P), f32),
            pltpu.VMEM((_K, _P, _P), f32),
            pltpu.VMEM((_P, 128), f32),
        ],
        compiler_params=pltpu.CompilerParams(
            dimension_semantics=("parallel",)),
    )(posp, post, oha, emb, EW, EB, sp, bx)

    outr = out.reshape(_B, 8, 128)
    hg = outr[:, 0, 0]
    means = outr[:, 1:1 + _NL, :_NEXP].sum(axis=0) / f32(_B * _NPG * _K)
    lb_layers = jnp.sum(means * means, axis=1) * _NEXP
    total_lb = jnp.sum(lb_layers) / _NL * jnp.float32(0.1)
    return hg, total_lb


# speedup vs baseline: 27.3860x; 1.6282x over previous
import jax
import jax.numpy as jnp
from jax.experimental import pallas as pl
from jax.experimental.pallas import tpu as pltpu

_B, _NPG, _K, _N, _D, _CUTOFF = 20, 500, 32, 10000, 128, 10.0
_NEXP = 4
_NL = 2
_P = 512
_ATP = 256


def _graph_body(posp_ref, post_ref, ohat_ref, embt_ref, ewt_ref, ebt_ref,
                sp_ref, out_ref, d2_ref, oh_ref, ewm_ref):
    f32 = jnp.float32
    i32 = jnp.int32
    lane_pi = jax.lax.broadcasted_iota(i32, (_P, _P), 1)
    sub_pi = jax.lax.broadcasted_iota(i32, (_P, _P), 0)

    bad = (lane_pi == sub_pi) | (sub_pi >= _NPG)
    d2 = jnp.where(bad, f32(1e9), f32(0.0))
    for c in range(3):
        diff = posp_ref[:, c:c + 1] - post_ref[c:c + 1, :]
        d2 = d2 + diff * diff
    d2_ref[...] = d2
    sub_pf = sub_pi.astype(f32)

    for k in range(_K):
        dd = d2_ref[...]
        m = jnp.min(dd, axis=0, keepdims=True)
        sidx = jnp.min(jnp.where(dd == m, sub_pf, f32(1e9)), axis=0,
                       keepdims=True)
        hit = sub_pf == sidx
        oh_ref[k] = hit.astype(f32)
        d2_ref[...] = jnp.where(hit, f32(3e9), dd)
        ewm_ref[k:k + 1, :] = jnp.sqrt(jnp.maximum(m, 0.0) + 1e-12)

    lane_e = jax.lax.broadcasted_iota(i32, (_K, _P), 1)
    emask = (lane_e < _NPG).astype(f32)
    ecnt = f32(_NPG * _K)

    ewm = ewm_ref[...]
    mean = jnp.sum(ewm * emask) / ecnt
    cen = ewm - sp_ref[2, 2] * mean
    var = jnp.sum(cen * cen * emask) / ecnt
    ea = sp_ref[2, 0] * cen * jax.lax.rsqrt(var + 1e-5) + sp_ref[2, 1]
    ewt = (_CUTOFF - ewm) / _CUTOFF

    hT = jnp.dot(embt_ref[...], ohat_ref[...], preferred_element_type=f32)

    lane_d = jax.lax.broadcasted_iota(i32, (_D, _P), 1)
    nmask = (lane_d < _NPG).astype(f32)

    attsums = []
    for l in range(_NL):
        logits = [sp_ref[l, 96 + e] + jnp.zeros((_K, _P), f32)
                  for e in range(_NEXP)]
        for j in range(16):
            h1 = jnp.maximum(ea * sp_ref[l, j] + sp_ref[l, 16 + j], 0.0)
            for e in range(_NEXP):
                logits[e] = logits[e] + h1 * sp_ref[l, 32 + j * 4 + e]
        mx = jnp.maximum(jnp.maximum(logits[0], logits[1]),
                         jnp.maximum(logits[2], logits[3]))
        ex = [jnp.exp(lg - mx) for lg in logits]
        den = ex[0] + ex[1] + ex[2] + ex[3]
        att = [v / den for v in ex]
        attsums.append([jnp.sum(a * emask) for a in att])
        att = [a * ewt for a in att]

        agg = [jnp.zeros((_D, _P), f32) for _ in range(_NEXP)]
        for k in range(_K):
            g = jnp.dot(hT, oh_ref[k], preferred_element_type=f32)
            for e in range(_NEXP):
                agg[e] = agg[e] + att[e][k:k + 1, :] * g

        hn = jnp.zeros((_D, _P), f32)
        for e in range(_NEXP):
            r = l * 8 + e * 2
            t = jnp.maximum(
                jnp.dot(ewt_ref[r], agg[e], preferred_element_type=f32)
                + ebt_ref[:, r:r + 1], 0.0)
            hn = hn + jnp.dot(ewt_ref[r + 1], t, preferred_element_type=f32) \
                + ebt_ref[:, r + 1:r + 2]
        hT = hn

        if l + 1 < _NL:
            mean_c = jnp.sum(hT * nmask, axis=1, keepdims=True) / f32(_NPG)
            cen_c = hT - ebt_ref[:, 18:19] * mean_c
            var_c = jnp.sum(cen_c * cen_c * nmask, axis=1,
                            keepdims=True) / f32(_NPG)
            hT = ebt_ref[:, 16:17] * cen_c * jax.lax.rsqrt(var_c + 1e-5) \
                + ebt_ref[:, 17:18]
            hT = jnp.tanh(hT)

    hg = jnp.sum(hT * nmask) / f32(_NPG * _D)

    sub8 = jax.lax.broadcasted_iota(i32, (8, 128), 0)
    lane8 = jax.lax.broadcasted_iota(i32, (8, 128), 1)
    o = jnp.where((sub8 == 0) & (lane8 == 0), hg, f32(0.0))
    for l in range(_NL):
        for e in range(_NEXP):
            o = jnp.where((sub8 == l + 1) & (lane8 == e), attsums[l][e], o)
    out_ref[...] = o


def kernel(pos, atom_type, batch, params):
    f32 = jnp.float32
    pos_r = pos.reshape(_B, _NPG, 3).astype(f32)
    posp = jnp.zeros((_B, _P, 128), f32).at[:, :_NPG, :3].set(pos_r)
    posp = posp.reshape(_B * _P, 128)
    post = jnp.zeros((_B, 128, _P), f32).at[:, :3, :_NPG].set(
        pos_r.transpose(0, 2, 1)).reshape(_B * 128, _P)
    ohat = jax.nn.one_hot(atom_type, _ATP, dtype=f32).reshape(_B, _NPG, _ATP)
    ohat = jnp.pad(ohat, ((0, 0), (0, _P - _NPG), (0, 0)))
    ohat = ohat.transpose(0, 2, 1).reshape(_B * _ATP, _P)
    embt = jnp.pad(params["atom_emb"].astype(f32).T, ((0, 0), (0, _ATP - 200)))

    layers = params["layers"]
    ew_rows, eb_cols = [], []
    for l in range(_NL):
        for e in range(_NEXP):
            exp = layers[l]["experts"][e]
            ew_rows += [exp["W1"].astype(f32).T, exp["W2"].astype(f32).T]
            eb_cols += [exp["b1"].astype(f32), exp["b2"].astype(f32)]
    EWT = jnp.stack(ew_rows)
    EBT = jnp.stack(eb_cols
                    + [layers[0]["gn_g"].astype(f32),
                       layers[0]["gn_b"].astype(f32),
                       layers[0]["gn_ms"].astype(f32)]
                    + [jnp.zeros((_D,), f32)] * 5, axis=1)

    sp = jnp.zeros((8, 128), f32)
    for l in range(_NL):
        L = layers[l]
        sp = sp.at[l, 0:16].set(L["eW1"].reshape(16).astype(f32))
        sp = sp.at[l, 16:32].set(L["eb1"].astype(f32))
        sp = sp.at[l, 32:96].set(L["eW2"].reshape(64).astype(f32))
        sp = sp.at[l, 96:100].set(L["eb2"].astype(f32))
    sp = sp.at[2, 0].set(params["dn_g"][0]) \
           .at[2, 1].set(params["dn_b"][0]) \
           .at[2, 2].set(params["dn_ms"][0])

    out = pl.pallas_call(
        _graph_body,
        grid=(_B,),
        in_specs=[
            pl.BlockSpec((_P, 128), lambda b: (b, 0)),
            pl.BlockSpec((128, _P), lambda b: (b, 0)),
            pl.BlockSpec((_ATP, _P), lambda b: (b, 0)),
            pl.BlockSpec((_D, _ATP), lambda b: (0, 0)),
            pl.BlockSpec((16, 128, 128), lambda b: (0, 0, 0)),
            pl.BlockSpec((_D, 24), lambda b: (0, 0)),
            pl.BlockSpec(memory_space=pltpu.SMEM),
        ],
        out_specs=pl.BlockSpec((8, 128), lambda b: (b, 0)),
        out_shape=jax.ShapeDtypeStruct((_B * 8, 128), f32),
        scratch_shapes=[
            pltpu.VMEM((_P, _P), f32),
            pltpu.VMEM((_K, _P, _P), f32),
            pltpu.VMEM((_K, _P), f32),
        ],
        compiler_params=pltpu.CompilerParams(
            dimension_semantics=("parallel",)),
    )(posp, post, ohat, embt, EWT, EBT, sp)

    outr = out.reshape(_B, 8, 128)
    hg = outr[:, 0, 0]
    means = outr[:, 1:1 + _NL, :_NEXP].sum(axis=0) / f32(_B * _NPG * _K)
    lb_layers = jnp.sum(means * means, axis=1) * _NEXP
    total_lb = jnp.sum(lb_layers) / _NL * jnp.float32(0.1)
    return hg, total_lb
